# Initial kernel scaffold; baseline (speedup 1.0000x reference)
import jax
import jax.numpy as jnp
from jax.experimental import pallas as pl


def kernel(input, labels, weightx, weighty):
    raise NotImplementedError("write your pallas kernel here")


# speedup vs baseline: 3628.5435x; 3628.5435x over previous
import functools

import jax
import jax.numpy as jnp
from jax import lax
from jax.experimental import pallas as pl
from jax.experimental.pallas import tpu as pltpu
from jax.experimental.pallas import tpu_sc as plsc

_L = 16


def _sc_body(nw, planes, m, c_dim, kp, x_hbm, wy_hbm, par_hbm, out_hbm,
             in_v, out_v, tab_v, par_v):
    wid = lax.axis_index("s") * 2 + lax.axis_index("c")
    pltpu.sync_copy(par_hbm, par_v)
    w_lo = par_v[pl.ds(0, _L)]
    w_hi = par_v[pl.ds(_L, _L)]
    inv_dx = par_v[pl.ds(2 * _L, _L)]
    nvec = m // _L
    per_w = planes // nw

    def plane_body(t, carry):
        p = wid * per_w + t
        c = lax.rem(p, c_dim)
        pltpu.sync_copy(wy_hbm.at[c], tab_v)
        pltpu.sync_copy(x_hbm.at[p], in_v)

        def vec_body(i, carry2):
            x = in_v[pl.ds(i * _L, _L)]
            x = jnp.minimum(jnp.maximum(x, w_lo), w_hi)
            tt = (x - w_lo) * inv_dx
            i0 = jnp.minimum(tt.astype(jnp.int32),
                             jnp.full((_L,), kp - 2, jnp.int32))
            frac = tt - i0.astype(jnp.float32)
            y0 = plsc.load_gather(tab_v, [i0])
            y1 = plsc.load_gather(tab_v, [i0 + jnp.full((_L,), 1, jnp.int32)])
            out_v[pl.ds(i * _L, _L)] = y0 + (y1 - y0) * frac
            return carry2

        lax.fori_loop(0, nvec, vec_body, 0)
        pltpu.sync_copy(out_v, out_hbm.at[p])
        return carry

    lax.fori_loop(0, per_w, plane_body, 0)


def kernel(input, labels, weightx, weighty):
    B, C, H, W = input.shape
    K = weightx.shape[0]
    P = B * C
    M = H * W
    NW = 32
    assert P % NW == 0 and M % _L == 0

    wy = weighty[labels]
    Kp = ((K + _L - 1) // _L) * _L
    wy_pad = jnp.pad(wy, ((0, 0), (0, Kp - K)))

    w_lo = weightx[0]
    w_hi = weightx[-1]
    inv_dx = (K - 1) / (w_hi - w_lo)
    par = jnp.concatenate([
        jnp.full((_L,), w_lo), jnp.full((_L,), w_hi), jnp.full((_L,), inv_dx)
    ]).astype(jnp.float32)

    xf = input.reshape(P, M)
    mesh = plsc.VectorSubcoreMesh(core_axis_name="c", subcore_axis_name="s")
    body = functools.partial(_sc_body, NW, P, M, C, K)
    out = pl.kernel(
        body,
        mesh=mesh,
        compiler_params=pltpu.CompilerParams(needs_layout_passes=False),
        out_type=jax.ShapeDtypeStruct((P, M), jnp.float32),
        scratch_types=[
            pltpu.VMEM((M,), jnp.float32),
            pltpu.VMEM((M,), jnp.float32),
            pltpu.VMEM((Kp,), jnp.float32),
            pltpu.VMEM((3 * _L,), jnp.float32),
        ],
    )(xf, wy_pad, par)
    return out.reshape(B, C, H, W)


# speedup vs baseline: 5810.9055x; 1.6014x over previous
import functools

import jax
import jax.numpy as jnp
from jax import lax
from jax.experimental import pallas as pl
from jax.experimental.pallas import tpu as pltpu
from jax.experimental.pallas import tpu_sc as plsc

_L = 16


def _sc_body(nw, planes, m, c_dim, kp, x_hbm, wy_hbm, par_hbm, out_hbm,
             in_v, out_v, tab_v, par_v):
    wid = lax.axis_index("s") * 2 + lax.axis_index("c")
    pltpu.sync_copy(par_hbm, par_v)
    w_lo = par_v[pl.ds(0, _L)]
    w_hi = par_v[pl.ds(_L, _L)]
    inv_dx = par_v[pl.ds(2 * _L, _L)]
    nvec = m // _L
    per_w = planes // nw

    def plane_body(t, carry):
        p = wid * per_w + t
        c = lax.rem(p, c_dim)
        pltpu.sync_copy(wy_hbm.at[c], tab_v)
        pltpu.sync_copy(x_hbm.at[p], in_v)

        @plsc.parallel_loop(0, m, step=_L, unroll=8)
        def vec_body(i):
            x = in_v[pl.ds(i, _L)]
            x = jnp.minimum(jnp.maximum(x, w_lo), w_hi)
            tt = (x - w_lo) * inv_dx
            i0 = jnp.minimum(tt.astype(jnp.int32),
                             jnp.full((_L,), kp - 2, jnp.int32))
            frac = tt - i0.astype(jnp.float32)
            y0 = plsc.load_gather(tab_v, [i0])
            y1 = plsc.load_gather(tab_v, [i0 + jnp.full((_L,), 1, jnp.int32)])
            out_v[pl.ds(i, _L)] = y0 + (y1 - y0) * frac
        pltpu.sync_copy(out_v, out_hbm.at[p])
        return carry

    lax.fori_loop(0, per_w, plane_body, 0)


def kernel(input, labels, weightx, weighty):
    B, C, H, W = input.shape
    K = weightx.shape[0]
    P = B * C
    M = H * W
    NW = 32
    assert P % NW == 0 and M % _L == 0

    wy = weighty[labels]
    Kp = ((K + _L - 1) // _L) * _L
    wy_pad = jnp.pad(wy, ((0, 0), (0, Kp - K)))

    w_lo = weightx[0]
    w_hi = weightx[-1]
    inv_dx = (K - 1) / (w_hi - w_lo)
    par = jnp.concatenate([
        jnp.full((_L,), w_lo), jnp.full((_L,), w_hi), jnp.full((_L,), inv_dx)
    ]).astype(jnp.float32)

    xf = input.reshape(P, M)
    mesh = plsc.VectorSubcoreMesh(core_axis_name="c", subcore_axis_name="s")
    body = functools.partial(_sc_body, NW, P, M, C, K)
    out = pl.kernel(
        body,
        mesh=mesh,
        compiler_params=pltpu.CompilerParams(needs_layout_passes=False),
        out_type=jax.ShapeDtypeStruct((P, M), jnp.float32),
        scratch_types=[
            pltpu.VMEM((M,), jnp.float32),
            pltpu.VMEM((M,), jnp.float32),
            pltpu.VMEM((Kp,), jnp.float32),
            pltpu.VMEM((3 * _L,), jnp.float32),
        ],
    )(xf, wy_pad, par)
    return out.reshape(B, C, H, W)


# speedup vs baseline: 7298.1644x; 1.2559x over previous
import functools

import numpy as np

import jax
import jax.numpy as jnp
from jax import lax
from jax.experimental import pallas as pl
from jax.experimental.pallas import tpu as pltpu
from jax.experimental.pallas import tpu_sc as plsc

_L = 16
_NW = 32


def _sc_body(planes, m, c_dim, kp, x_hbm, wy_hbm, par_hbm, out_hbm,
             in0, in1, ou0, ou1, tabs_v, par_v, si0, si1, so0, so1):
    wid = lax.axis_index("s") * 2 + lax.axis_index("c")
    pltpu.sync_copy(par_hbm, par_v)
    pltpu.sync_copy(wy_hbm, tabs_v)
    w_lo = par_v[pl.ds(0, _L)]
    inv_dx = par_v[pl.ds(_L, _L)]
    tmax = float(np.nextafter(np.float32(kp - 1), np.float32(0.0)))
    half = m // 2
    per_w = planes // _NW
    nchunk = 2 * per_w

    def chunk_coords(k):
        p = wid * per_w + k // 2
        off = (k % 2) * half
        return p, off

    def start_in(k, buf, sem):
        p, off = chunk_coords(k)
        return pltpu.async_copy(x_hbm.at[p, pl.ds(off, half)], buf, sem)

    def compute(k, buf, obuf):
        p, off = chunk_coords(k)
        c = lax.rem(p, c_dim)
        tab = tabs_v.at[c]

        @plsc.parallel_loop(0, half, step=_L, unroll=8)
        def vec_body(i):
            x = buf[pl.ds(i, _L)]
            tt = (x - w_lo) * inv_dx
            tt = jnp.maximum(tt, jnp.zeros((_L,), jnp.float32))
            tt = jnp.minimum(tt, jnp.full((_L,), tmax, jnp.float32))
            i0 = tt.astype(jnp.int32)
            fr = tt - i0.astype(jnp.float32)
            y0 = plsc.load_gather(tab, [i0])
            y1 = plsc.load_gather(tab, [i0 + jnp.full((_L,), 1, jnp.int32)])
            obuf[pl.ds(i, _L)] = y0 + (y1 - y0) * fr

    def start_out(k, obuf, sem):
        p, off = chunk_coords(k)
        return pltpu.async_copy(obuf, out_hbm.at[p, pl.ds(off, half)], sem)

    def wait_out(k, obuf, sem):
        p, off = chunk_coords(k)
        pltpu.make_async_copy(obuf, out_hbm.at[p, pl.ds(off, half)], sem).wait()

    start_in(0, in0, si0)

    def pair_body(t, carry):
        ka = 2 * t
        kb = 2 * t + 1
        start_in(kb, in1, si1)
        pltpu.make_async_copy(x_hbm.at[0, pl.ds(0, half)], in0, si0).wait()

        @pl.when(t > 0)
        def _():
            wait_out(ka, ou0, so0)

        compute(ka, in0, ou0)
        start_out(ka, ou0, so0)

        @pl.when(t + 1 < per_w)
        def _():
            start_in(ka + 2, in0, si0)

        pltpu.make_async_copy(x_hbm.at[0, pl.ds(0, half)], in1, si1).wait()

        @pl.when(t > 0)
        def _():
            wait_out(kb, ou1, so1)

        compute(kb, in1, ou1)
        start_out(kb, ou1, so1)
        return carry

    lax.fori_loop(0, per_w, pair_body, 0)
    wait_out(nchunk - 2, ou0, so0)
    wait_out(nchunk - 1, ou1, so1)


def kernel(input, labels, weightx, weighty):
    B, C, H, W = input.shape
    K = weightx.shape[0]
    P = B * C
    M = H * W
    assert P % _NW == 0 and (M // 2) % _L == 0

    wy = weighty[labels]
    Kp = ((K + _L - 1) // _L) * _L
    wy_pad = jnp.pad(wy, ((0, 0), (0, Kp - K)))

    w_lo = weightx[0]
    inv_dx = (K - 1) / (weightx[-1] - w_lo)
    par = jnp.concatenate(
        [jnp.full((_L,), w_lo), jnp.full((_L,), inv_dx)]
    ).astype(jnp.float32)

    xf = input.reshape(P, M)
    mesh = plsc.VectorSubcoreMesh(core_axis_name="c", subcore_axis_name="s")
    body = functools.partial(_sc_body, P, M, C, K)
    out = pl.kernel(
        body,
        mesh=mesh,
        compiler_params=pltpu.CompilerParams(needs_layout_passes=False),
        out_type=jax.ShapeDtypeStruct((P, M), jnp.float32),
        scratch_types=[
            pltpu.VMEM((M // 2,), jnp.float32),
            pltpu.VMEM((M // 2,), jnp.float32),
            pltpu.VMEM((M // 2,), jnp.float32),
            pltpu.VMEM((M // 2,), jnp.float32),
            pltpu.VMEM((C, Kp), jnp.float32),
            pltpu.VMEM((2 * _L,), jnp.float32),
            pltpu.SemaphoreType.DMA,
            pltpu.SemaphoreType.DMA,
            pltpu.SemaphoreType.DMA,
            pltpu.SemaphoreType.DMA,
        ],
    )(xf, wy_pad, par)
    return out.reshape(B, C, H, W)


# speedup vs baseline: 7826.5046x; 1.0724x over previous
import functools

import numpy as np

import jax
import jax.numpy as jnp
from jax import lax
from jax.experimental import pallas as pl
from jax.experimental.pallas import tpu as pltpu
from jax.experimental.pallas import tpu_sc as plsc

_L = 16
_NW = 32


def _sc_body(planes, m, c_dim, kp, x_hbm, wy_hbm, par_hbm, out_hbm,
             in0, in1, ou0, ou1, tabs_v, ta_v, tb_v, par_v,
             si0, si1, so0, so1):
    wid = lax.axis_index("s") * 2 + lax.axis_index("c")
    pltpu.sync_copy(par_hbm, par_v)
    w_lo = par_v[pl.ds(0, _L)]
    inv_dx = par_v[pl.ds(_L, _L)]
    tmax = float(np.nextafter(np.float32(kp - 1), np.float32(0.0)))
    half = m // 2
    per_w = planes // _NW
    kpad = ta_v.shape[1]

    c_base = lax.rem(wid * per_w, c_dim)
    pltpu.sync_copy(wy_hbm.at[pl.ds(c_base * kpad, per_w * kpad)], tabs_v)

    @plsc.parallel_loop(0, per_w, step=1)
    def prep_row(r):
        base = r * kpad
        for j in range(kpad // _L):
            kidx = lax.iota(jnp.int32, _L) + jnp.full((_L,), j * _L, jnp.int32)
            y0 = tabs_v[pl.ds(base + j * _L, _L)]
            idx1 = jnp.minimum(kidx + jnp.full((_L,), 1, jnp.int32),
                               jnp.full((_L,), kpad - 1, jnp.int32))
            y1 = plsc.load_gather(tabs_v, [idx1 + jnp.full((_L,), 1, jnp.int32) * base])
            dy = y1 - y0
            ta_v[r, pl.ds(j * _L, _L)] = y0 - kidx.astype(jnp.float32) * dy
            tb_v[r, pl.ds(j * _L, _L)] = dy

    def chunk_coords(k):
        p = wid * per_w + k // 2
        off = (k % 2) * half
        return p, off

    def start_in(k, buf, sem):
        p, off = chunk_coords(k)
        return pltpu.async_copy(x_hbm.at[p, pl.ds(off, half)], buf, sem)

    def compute(k, buf, obuf):
        ta = ta_v.at[k // 2]
        tb = tb_v.at[k // 2]

        @plsc.parallel_loop(0, half, step=_L, unroll=8)
        def vec_body(i):
            x = buf[pl.ds(i, _L)]
            tt = (x - w_lo) * inv_dx
            tt = jnp.maximum(tt, jnp.zeros((_L,), jnp.float32))
            tt = jnp.minimum(tt, jnp.full((_L,), tmax, jnp.float32))
            i0 = tt.astype(jnp.int32)
            a = plsc.load_gather(ta, [i0])
            b = plsc.load_gather(tb, [i0])
            obuf[pl.ds(i, _L)] = a + b * tt

    def start_out(k, obuf, sem):
        p, off = chunk_coords(k)
        return pltpu.async_copy(obuf, out_hbm.at[p, pl.ds(off, half)], sem)

    def wait_out(k, obuf, sem):
        p, off = chunk_coords(k)
        pltpu.make_async_copy(obuf, out_hbm.at[p, pl.ds(off, half)], sem).wait()

    start_in(0, in0, si0)

    def pair_body(t, carry):
        ka = 2 * t
        kb = 2 * t + 1
        start_in(kb, in1, si1)
        pltpu.make_async_copy(x_hbm.at[0, pl.ds(0, half)], in0, si0).wait()

        @pl.when(t > 0)
        def _():
            wait_out(ka, ou0, so0)

        compute(ka, in0, ou0)
        start_out(ka, ou0, so0)

        @pl.when(t + 1 < per_w)
        def _():
            start_in(ka + 2, in0, si0)

        pltpu.make_async_copy(x_hbm.at[0, pl.ds(0, half)], in1, si1).wait()

        @pl.when(t > 0)
        def _():
            wait_out(kb, ou1, so1)

        compute(kb, in1, ou1)
        start_out(kb, ou1, so1)
        return carry

    lax.fori_loop(0, per_w, pair_body, 0)
    wait_out(2 * per_w - 2, ou0, so0)
    wait_out(2 * per_w - 1, ou1, so1)


def kernel(input, labels, weightx, weighty):
    B, C, H, W = input.shape
    K = weightx.shape[0]
    P = B * C
    M = H * W
    assert P % _NW == 0 and (M // 2) % _L == 0
    per_w = P // _NW
    assert C % per_w == 0

    wy = weighty[labels]
    Kp = ((K + _L - 1) // _L) * _L
    wy_pad = jnp.pad(wy, ((0, 0), (0, Kp - K))).reshape(-1)

    w_lo = weightx[0]
    inv_dx = (K - 1) / (weightx[-1] - w_lo)
    par = jnp.concatenate(
        [jnp.full((_L,), w_lo), jnp.full((_L,), inv_dx)]
    ).astype(jnp.float32)

    xf = input.reshape(P, M)
    mesh = plsc.VectorSubcoreMesh(core_axis_name="c", subcore_axis_name="s")
    body = functools.partial(_sc_body, P, M, C, K)
    out = pl.kernel(
        body,
        mesh=mesh,
        compiler_params=pltpu.CompilerParams(needs_layout_passes=False),
        out_type=jax.ShapeDtypeStruct((P, M), jnp.float32),
        scratch_types=[
            pltpu.VMEM((M // 2,), jnp.float32),
            pltpu.VMEM((M // 2,), jnp.float32),
            pltpu.VMEM((M // 2,), jnp.float32),
            pltpu.VMEM((M // 2,), jnp.float32),
            pltpu.VMEM((P // _NW * Kp,), jnp.float32),
            pltpu.VMEM((P // _NW, Kp), jnp.float32),
            pltpu.VMEM((P // _NW, Kp), jnp.float32),
            pltpu.VMEM((2 * _L,), jnp.float32),
            pltpu.SemaphoreType.DMA,
            pltpu.SemaphoreType.DMA,
            pltpu.SemaphoreType.DMA,
            pltpu.SemaphoreType.DMA,
        ],
    )(xf, wy_pad, par)
    return out.reshape(B, C, H, W)


# speedup vs baseline: 17790.4579x; 2.2731x over previous
import functools

import numpy as np

import jax
import jax.numpy as jnp
from jax import lax
from jax.experimental import pallas as pl
from jax.experimental.pallas import tpu as pltpu
from jax.experimental.pallas import tpu_sc as plsc

_L = 16
_NW = 32


def _sc_body(shape, kp, x_hbm, wy_hbm, par_hbm, out_hbm,
             in0, in1, ou0, ou1, tabs_v, ta_v, tb_v, par_v,
             si0, si1, so0, so1):
    b_dim, c_dim, h_dim, w_dim = shape
    planes = b_dim * c_dim
    wid = lax.axis_index("s") * 2 + lax.axis_index("c")
    pltpu.sync_copy(par_hbm, par_v)
    w_lo = par_v[pl.ds(0, _L)]
    inv_dx = par_v[pl.ds(_L, _L)]
    tmax = float(np.nextafter(np.float32(kp - 1), np.float32(0.0)))
    hh = h_dim // 2
    per_w = planes // _NW
    kpad = ta_v.shape[1]
    nvec_w = w_dim // _L

    c_base = lax.rem(wid * per_w, c_dim)
    pltpu.sync_copy(wy_hbm.at[pl.ds(c_base * kpad, per_w * kpad)], tabs_v)

    @plsc.parallel_loop(0, per_w, step=1)
    def prep_row(r):
        base = r * kpad
        for j in range(kpad // _L):
            kidx = lax.iota(jnp.int32, _L) + jnp.full((_L,), j * _L, jnp.int32)
            y0 = tabs_v[pl.ds(base + j * _L, _L)]
            idx1 = jnp.minimum(kidx + jnp.full((_L,), 1, jnp.int32),
                               jnp.full((_L,), kpad - 1, jnp.int32))
            y1 = plsc.load_gather(tabs_v, [idx1 + jnp.full((_L,), 1, jnp.int32) * base])
            dy = y1 - y0
            ta_v[r, pl.ds(j * _L, _L)] = y0 - kidx.astype(jnp.float32) * dy
            tb_v[r, pl.ds(j * _L, _L)] = dy

    def chunk_coords(k):
        p = wid * per_w + k // 2
        b = p // c_dim
        c = lax.rem(p, c_dim)
        r0 = (k % 2) * hh
        return b, c, r0

    def start_in(k, buf, sem):
        b, c, r0 = chunk_coords(k)
        return pltpu.async_copy(x_hbm.at[b, c, pl.ds(r0, hh)], buf, sem)

    def compute(k, buf, obuf):
        ta = ta_v.at[k // 2]
        tb = tb_v.at[k // 2]

        @plsc.parallel_loop(0, hh, step=1, unroll=2)
        def row_body(r):
            for j in range(nvec_w):
                x = buf[r, pl.ds(j * _L, _L)]
                tt = (x - w_lo) * inv_dx
                tt = jnp.maximum(tt, jnp.zeros((_L,), jnp.float32))
                tt = jnp.minimum(tt, jnp.full((_L,), tmax, jnp.float32))
                i0 = tt.astype(jnp.int32)
                a = plsc.load_gather(ta, [i0])
                b = plsc.load_gather(tb, [i0])
                obuf[r, pl.ds(j * _L, _L)] = a + b * tt

    def start_out(k, obuf, sem):
        b, c, r0 = chunk_coords(k)
        return pltpu.async_copy(obuf, out_hbm.at[b, c, pl.ds(r0, hh)], sem)

    def wait_out(k, obuf, sem):
        b, c, r0 = chunk_coords(k)
        pltpu.make_async_copy(obuf, out_hbm.at[b, c, pl.ds(r0, hh)], sem).wait()

    start_in(0, in0, si0)

    def pair_body(t, carry):
        ka = 2 * t
        kb = 2 * t + 1
        start_in(kb, in1, si1)
        pltpu.make_async_copy(x_hbm.at[0, 0, pl.ds(0, hh)], in0, si0).wait()

        @pl.when(t > 0)
        def _():
            wait_out(ka, ou0, so0)

        compute(ka, in0, ou0)
        start_out(ka, ou0, so0)

        @pl.when(t + 1 < per_w)
        def _():
            start_in(ka + 2, in0, si0)

        pltpu.make_async_copy(x_hbm.at[0, 0, pl.ds(0, hh)], in1, si1).wait()

        @pl.when(t > 0)
        def _():
            wait_out(kb, ou1, so1)

        compute(kb, in1, ou1)
        start_out(kb, ou1, so1)
        return carry

    lax.fori_loop(0, per_w, pair_body, 0)
    wait_out(2 * per_w - 2, ou0, so0)
    wait_out(2 * per_w - 1, ou1, so1)


def kernel(input, labels, weightx, weighty):
    B, C, H, W = input.shape
    K = weightx.shape[0]
    P = B * C
    assert P % _NW == 0 and W % _L == 0 and H % 2 == 0
    per_w = P // _NW
    assert C % per_w == 0

    wy = weighty[labels]
    Kp = ((K + _L - 1) // _L) * _L
    wy_pad = jnp.pad(wy, ((0, 0), (0, Kp - K))).reshape(-1)

    w_lo = weightx[0]
    inv_dx = (K - 1) / (weightx[-1] - w_lo)
    par = jnp.concatenate(
        [jnp.full((_L,), w_lo), jnp.full((_L,), inv_dx)]
    ).astype(jnp.float32)

    mesh = plsc.VectorSubcoreMesh(core_axis_name="c", subcore_axis_name="s")
    body = functools.partial(_sc_body, (B, C, H, W), K)
    return pl.kernel(
        body,
        mesh=mesh,
        compiler_params=pltpu.CompilerParams(needs_layout_passes=False),
        out_type=jax.ShapeDtypeStruct((B, C, H, W), jnp.float32),
        scratch_types=[
            pltpu.VMEM((H // 2, W), jnp.float32),
            pltpu.VMEM((H // 2, W), jnp.float32),
            pltpu.VMEM((H // 2, W), jnp.float32),
            pltpu.VMEM((H // 2, W), jnp.float32),
            pltpu.VMEM((P // _NW * Kp,), jnp.float32),
            pltpu.VMEM((P // _NW, Kp), jnp.float32),
            pltpu.VMEM((P // _NW, Kp), jnp.float32),
            pltpu.VMEM((2 * _L,), jnp.float32),
            pltpu.SemaphoreType.DMA,
            pltpu.SemaphoreType.DMA,
            pltpu.SemaphoreType.DMA,
            pltpu.SemaphoreType.DMA,
        ],
    )(input, wy_pad, par)
